# baseline (speedup 1.0000x reference)
import functools

import jax
import jax.numpy as jnp
from jax import lax
from jax.experimental import pallas as pl
from jax.experimental.pallas import tpu as pltpu


def _fused_kernel(x_ref, w_ref, gamma_ref, beta_ref, o_ref,
                  ws_s, shift_s, *, n, bsz, inv_m, eps):
    i = pl.program_id(0)

    @pl.when(i == 0)
    def _stats_and_glue():
        x0 = x_ref[0]
        gram = lax.dot_general(x0, x0, (((1,), (1,)), ((), ())),
                               preferred_element_type=jnp.float32)
        xacc = x0.astype(jnp.float32)
        for m in range(1, n):
            xm = x_ref[m]
            gram = gram + lax.dot_general(xm, xm, (((1,), (1,)), ((), ())),
                                          preferred_element_type=jnp.float32)
            xacc = xacc + xm.astype(jnp.float32)
        sx = jnp.sum(xacc, axis=-1, keepdims=True)
        w = w_ref[...]
        mean = jnp.dot(w, sx, preferred_element_type=jnp.float32) * inv_m
        wg = jnp.dot(w, gram, preferred_element_type=jnp.float32)
        sumsq = jnp.sum(wg * w, axis=-1, keepdims=True)
        var = jnp.maximum(sumsq * inv_m - mean * mean, 0.0)
        scale = gamma_ref[...] * lax.rsqrt(var + eps)
        shift_s[...] = beta_ref[...] - mean * scale
        ws_s[...] = (w * scale).astype(jnp.bfloat16)

    ws = ws_s[...]
    sh = shift_s[...]
    for j in range(bsz):
        y = jnp.dot(ws, x_ref[i * bsz + j],
                    preferred_element_type=jnp.float32) + sh
        o_ref[j] = jnp.maximum(y, 0.0).astype(jnp.bfloat16)


def kernel(x, w, b, gamma, beta):
    del b
    eps = 1e-5
    N, Cin, D, H, W = x.shape
    Cout = w.shape[0]
    S = D * H * W
    M = N * S

    xbf = x.reshape(N, Cin, S).astype(jnp.bfloat16)

    B = 2 if N % 2 == 0 else 1
    NB = N // B

    body = functools.partial(_fused_kernel, n=N, bsz=B, inv_m=1.0 / M, eps=eps)
    outb = pl.pallas_call(
        body,
        grid=(NB,),
        in_specs=[pl.BlockSpec(memory_space=pltpu.MemorySpace.VMEM),
                  pl.BlockSpec((Cout, Cin), lambda i: (0, 0)),
                  pl.BlockSpec((Cout, 1), lambda i: (0, 0)),
                  pl.BlockSpec((Cout, 1), lambda i: (0, 0))],
        out_specs=pl.BlockSpec((B, Cout, S), lambda i: (i, 0, 0)),
        out_shape=jax.ShapeDtypeStruct((N, Cout, S), jnp.bfloat16),
        scratch_shapes=[pltpu.VMEM((Cout, Cin), jnp.bfloat16),
                        pltpu.VMEM((Cout, 1), jnp.float32)],
        compiler_params=pltpu.CompilerParams(
            dimension_semantics=("arbitrary",),
            vmem_limit_bytes=28 << 20),
    )(xbf, w, gamma.reshape(Cout, 1), beta.reshape(Cout, 1))

    return outb.astype(jnp.float32).reshape(N, Cout, D, H, W)
